# Initial kernel scaffold; baseline (speedup 1.0000x reference)
import jax
import jax.numpy as jnp
from jax.experimental import pallas as pl


def kernel(x, edge_index, edge_type, batch, comp1, basis1, root1, bias1, comp2, basis2, root2, bias2, bn_gamma, bn_beta, fc1_w, fc1_b, fc2_w, fc2_b):
    raise NotImplementedError("write your pallas kernel here")


# speedup vs baseline: 9.2477x; 9.2477x over previous
import functools

import jax
import jax.numpy as jnp
from jax import lax
from jax.experimental import pallas as pl
from jax.experimental.pallas import tpu as pltpu
from jax.experimental.pallas import tpu_sc as plsc

N_NODES = 10000
N_EDGES = 640000
NUM_REL = 4
NUM_BASES = 30
IN_DIM = 86
HID = 128
N_GRAPHS = 64
N_CLASSES = 18

NC = 2
NS = 16
CHUNK = 128
SB = 16
NSB = 20
NCHUNK = SB * NSB
EPT = NCHUNK * CHUNK
E_PAD = EPT * NS
ACC_ROWS = 40064
RPT = ACC_ROWS // NS
DUMMY_ROW = NUM_REL * N_NODES

NB = 2000
NBLK = N_NODES // NB


def _make_agg(width, passes_per_core):
    ntab = NC * passes_per_core

    def body(src_ref, cmb_ref, *rest):
        tables = rest[:ntab]
        zeros_ref = rest[ntab]
        out_ref = rest[ntab + 1]
        acc, idx_s, idx_c, rows = rest[ntab + 2:]
        c = lax.axis_index("c")
        s = lax.axis_index("s")

        def run_pass(table_ref, p):
            pltpu.sync_copy(zeros_ref, acc.at[pl.ds(s * RPT, RPT)])
            plsc.subcore_barrier()

            def block(b, carry):
                pltpu.sync_copy(src_ref.at[s].at[b], idx_s)
                pltpu.sync_copy(cmb_ref.at[s].at[b], idx_c)

                def step(g, carry2):
                    pltpu.sync_copy(table_ref.at[idx_s.at[g]], rows)
                    pltpu.sync_copy(rows, acc.at[idx_c.at[g]], add=True)
                    return carry2

                lax.fori_loop(0, SB, step, 0)
                return carry

            lax.fori_loop(0, NSB, block, 0)
            plsc.subcore_barrier()
            pltpu.sync_copy(acc.at[pl.ds(s * RPT, RPT)],
                            out_ref.at[p].at[pl.ds(s * RPT, RPT)])

        for j in range(passes_per_core):
            for cc in range(NC):
                p = cc * passes_per_core + j

                @pl.when(c == cc)
                def _(p=p):
                    run_pass(tables[p], p)

    return pl.kernel(
        body,
        out_type=jax.ShapeDtypeStruct((ntab, ACC_ROWS, width), jnp.float32),
        mesh=plsc.VectorSubcoreMesh(core_axis_name="c", subcore_axis_name="s"),
        compiler_params=pltpu.CompilerParams(use_tc_tiling_on_sc=False),
        scratch_types=[
            pltpu.VMEM_SHARED((ACC_ROWS, width), jnp.float32),
            pltpu.VMEM((SB, CHUNK), jnp.int32),
            pltpu.VMEM((SB, CHUNK), jnp.int32),
            pltpu.VMEM((CHUNK, width), jnp.float32),
        ],
    )


_agg48 = _make_agg(48, 1)
_agg32 = _make_agg(32, 2)


def _cmb_body(et_ref, dst_ref, out_ref):
    out_ref[...] = et_ref[...] * N_NODES + dst_ref[...]


def _cmb(et2, dst2):
    return pl.pallas_call(
        _cmb_body,
        out_shape=jax.ShapeDtypeStruct(et2.shape, jnp.int32),
    )(et2, dst2)


def _wexp_body(c1_ref, b1_ref, c2_ref, b2_ref, w1_ref, w2_ref):
    w1_ref[...] = jnp.dot(c1_ref[...], b1_ref[...],
                          preferred_element_type=jnp.float32)
    w2_ref[...] = jnp.dot(c2_ref[...], b2_ref[...],
                          preferred_element_type=jnp.float32)


def _wexp(comp1, basis1f, comp2, basis2f):
    return pl.pallas_call(
        _wexp_body,
        out_shape=(
            jax.ShapeDtypeStruct((NUM_REL, IN_DIM * HID), jnp.float32),
            jax.ShapeDtypeStruct((NUM_REL, HID * HID), jnp.float32),
        ),
    )(comp1, basis1f, comp2, basis2f)


def _dense1_body(xp_ref, root_ref, a0_ref, a1_ref, w_ref, h_ref):
    r = pl.program_id(1)

    @pl.when(r == 0)
    def _():
        h_ref[...] = jnp.dot(xp_ref[...], root_ref[...],
                             preferred_element_type=jnp.float32)

    @pl.when(r > 0)
    def _():
        a0 = a0_ref[0]
        a1 = a1_ref[0]
        cnt = jnp.maximum(a1[:, 38:39], 1.0)
        w = w_ref[0]
        part = (jnp.dot(a0, w[:48, :], preferred_element_type=jnp.float32)
                + jnp.dot(a1, w[48:, :], preferred_element_type=jnp.float32))
        h_ref[...] = h_ref[...] + part / cnt

    @pl.when(r == NUM_REL)
    def _():
        h_ref[...] = jnp.maximum(h_ref[...], 0.0)


def _dense1(xp, root1p, acc1, w1p):
    blk = N_NODES // NB
    return pl.pallas_call(
        _dense1_body,
        grid=(NBLK, NUM_REL + 1),
        in_specs=[
            pl.BlockSpec((NB, 96), lambda i, r: (i, 0)),
            pl.BlockSpec((96, HID), lambda i, r: (0, 0)),
            pl.BlockSpec((1, NB, 48),
                         lambda i, r: (0, jnp.maximum(r - 1, 0) * blk + i, 0)),
            pl.BlockSpec((1, NB, 48),
                         lambda i, r: (1, jnp.maximum(r - 1, 0) * blk + i, 0)),
            pl.BlockSpec((1, 96, HID), lambda i, r: (jnp.maximum(r - 1, 0), 0, 0)),
        ],
        out_specs=pl.BlockSpec((NB, HID), lambda i, r: (i, 0)),
        out_shape=jax.ShapeDtypeStruct((N_NODES, HID), jnp.float32),
    )(xp, root1p, acc1, acc1, w1p)


def _dense2_body(h_ref, root_ref, bias_ref, a20_ref, a21_ref, a22_ref, a23_ref,
                 a1_ref, w_ref, b_ref, g_ref, h2):
    i = pl.program_id(0)
    r = pl.program_id(1)

    @pl.when(r == 0)
    def _():
        h2[...] = jnp.dot(h_ref[...], root_ref[...],
                          preferred_element_type=jnp.float32) + bias_ref[...]

    @pl.when(r > 0)
    def _():
        w = w_ref[0]
        part = (jnp.dot(a20_ref[0], w[0:32, :], preferred_element_type=jnp.float32)
                + jnp.dot(a21_ref[0], w[32:64, :], preferred_element_type=jnp.float32)
                + jnp.dot(a22_ref[0], w[64:96, :], preferred_element_type=jnp.float32)
                + jnp.dot(a23_ref[0], w[96:128, :], preferred_element_type=jnp.float32))
        cnt = jnp.maximum(a1_ref[0][:, 38:39], 1.0)
        h2[...] = h2[...] + part / cnt

    @pl.when(r == NUM_REL)
    def _():
        b = b_ref[0]
        oh = (lax.broadcasted_iota(jnp.int32, (N_GRAPHS, NB), 0)
              == jnp.broadcast_to(b, (N_GRAPHS, NB))).astype(jnp.float32)
        contrib = jnp.dot(oh, h2[...], preferred_element_type=jnp.float32)
        g_ref[...] = jnp.where(i == 0, contrib, g_ref[...] + contrib)


def _dense2(h, root2, bias2, acc2, acc1, w2, batch2):
    blk = N_NODES // NB

    def amap(p):
        return lambda i, r: (p, jnp.maximum(r - 1, 0) * blk + i, 0)

    return pl.pallas_call(
        _dense2_body,
        grid=(NBLK, NUM_REL + 1),
        in_specs=[
            pl.BlockSpec((NB, HID), lambda i, r: (i, 0)),
            pl.BlockSpec((HID, HID), lambda i, r: (0, 0)),
            pl.BlockSpec((1, HID), lambda i, r: (0, 0)),
            pl.BlockSpec((1, NB, 32), amap(0)),
            pl.BlockSpec((1, NB, 32), amap(1)),
            pl.BlockSpec((1, NB, 32), amap(2)),
            pl.BlockSpec((1, NB, 32), amap(3)),
            pl.BlockSpec((1, NB, 48),
                         lambda i, r: (1, jnp.maximum(r - 1, 0) * blk + i, 0)),
            pl.BlockSpec((1, HID, HID), lambda i, r: (jnp.maximum(r - 1, 0), 0, 0)),
            pl.BlockSpec((1, 1, NB), lambda i, r: (i, 0, 0)),
        ],
        out_specs=pl.BlockSpec((N_GRAPHS, HID), lambda i, r: (0, 0)),
        out_shape=jax.ShapeDtypeStruct((N_GRAPHS, HID), jnp.float32),
        scratch_shapes=[pltpu.VMEM((NB, HID), jnp.float32)],
    )(h, root2, bias2, acc2, acc2, acc2, acc2, acc1, w2, batch2)


def _head_body(g_ref, gam_ref, bet_ref, w1_ref, b1_ref, w2_ref, b2_ref, out_ref):
    g = g_ref[...]
    mean = jnp.mean(g, axis=0, keepdims=True)
    var = jnp.mean((g - mean) * (g - mean), axis=0, keepdims=True)
    gn = (g - mean) / jnp.sqrt(var + 1e-5) * gam_ref[...] + bet_ref[...]
    hh = jnp.dot(gn, w1_ref[...], preferred_element_type=jnp.float32) + b1_ref[...]
    hh = jnp.maximum(hh, 0.0)
    logits = jnp.dot(hh, w2_ref[...], preferred_element_type=jnp.float32) + b2_ref[...]
    col = lax.broadcasted_iota(jnp.int32, (N_GRAPHS, HID), 1)
    logits = jnp.where(col < N_CLASSES, logits, -1e30)
    m = jnp.max(logits, axis=1, keepdims=True)
    lse = m + jnp.log(jnp.sum(jnp.exp(logits - m), axis=1, keepdims=True))
    out_ref[...] = logits - lse


def _head(g, gam, bet, w1, b1, w2p, b2p):
    return pl.pallas_call(
        _head_body,
        out_shape=jax.ShapeDtypeStruct((N_GRAPHS, HID), jnp.float32),
    )(g, gam, bet, w1, b1, w2p, b2p)


@jax.jit
def kernel(x, edge_index, edge_type, batch, comp1, basis1, root1, bias1,
           comp2, basis2, root2, bias2, bn_gamma, bn_beta,
           fc1_w, fc1_b, fc2_w, fc2_b):
    src = edge_index[0].astype(jnp.int32)
    dst = edge_index[1].astype(jnp.int32)
    et = edge_type.astype(jnp.int32)

    cmb = _cmb(et.reshape(5000, 128), dst.reshape(5000, 128)).reshape(-1)

    pad = E_PAD - N_EDGES
    srcp = jnp.concatenate([src, jnp.zeros((pad,), jnp.int32)])
    cmbp = jnp.concatenate([cmb, jnp.full((pad,), DUMMY_ROW, jnp.int32)])
    src3 = srcp.reshape(NS, NSB, SB, CHUNK)
    cmb3 = cmbp.reshape(NS, NSB, SB, CHUNK)

    w1f, w2f = _wexp(comp1, basis1.reshape(NUM_BASES, IN_DIM * HID),
                     comp2, basis2.reshape(NUM_BASES, HID * HID))
    w1 = w1f.reshape(NUM_REL, IN_DIM, HID)
    w1p = jnp.concatenate(
        [w1, jnp.zeros((NUM_REL, 96 - IN_DIM, HID), jnp.float32)], axis=1)
    w2 = w2f.reshape(NUM_REL, HID, HID)

    xp = jnp.concatenate(
        [x, jnp.ones((N_NODES, 1), jnp.float32),
         jnp.zeros((N_NODES, 96 - IN_DIM - 1), jnp.float32)], axis=1)
    root1p = jnp.concatenate(
        [root1, bias1[None, :], jnp.zeros((96 - IN_DIM - 1, HID), jnp.float32)],
        axis=0)

    z48 = jnp.zeros((RPT, 48), jnp.float32)
    acc1 = _agg48(src3, cmb3, xp[:, :48], xp[:, 48:96], z48)

    h = _dense1(xp, root1p, acc1, w1p)

    z32 = jnp.zeros((RPT, 32), jnp.float32)
    acc2 = _agg32(src3, cmb3, h[:, 0:32], h[:, 32:64], h[:, 64:96],
                  h[:, 96:128], z32)

    batch2 = batch.astype(jnp.int32).reshape(NBLK, 1, NB)
    g = _dense2(h, root2, bias2[None, :], acc2, acc1, w2, batch2)

    fc2_wp = jnp.concatenate(
        [fc2_w, jnp.zeros((HID, HID - N_CLASSES), jnp.float32)], axis=1)
    fc2_bp = jnp.concatenate(
        [fc2_b, jnp.zeros((HID - N_CLASSES,), jnp.float32)])[None, :]
    out = _head(g, bn_gamma[None, :], bn_beta[None, :],
                fc1_w, fc1_b[None, :], fc2_wp, fc2_bp)
    return out[:, :N_CLASSES]
